# Initial kernel scaffold; baseline (speedup 1.0000x reference)
import jax
import jax.numpy as jnp
from jax.experimental import pallas as pl


def kernel(A1, A2, A3, B1, B2, B3, peptide, W_a1, W_a2, W_a3, W_b1, W_b2, W_b3, W_peptide):
    raise NotImplementedError("write your pallas kernel here")


# speedup vs baseline: 1.4566x; 1.4566x over previous
import functools

import jax
import jax.numpy as jnp
from jax import lax
from jax.experimental import pallas as pl
from jax.experimental.pallas import tpu as pltpu
from jax.experimental.pallas import tpu_sc as plsc

VOCAB = 33
EMB = 128
B = 16384
L = 20
N = B * L
NUM_TABLES = 7

NC = 2
NS = 16
NW = NC * NS
PER_W = N // NW
CHUNK = 128
NCHUNK = PER_W // CHUNK


def _sc_lookup(*args):
    mesh = plsc.VectorSubcoreMesh(core_axis_name="c", subcore_axis_name="s")
    out_type = tuple(
        jax.ShapeDtypeStruct((N, EMB), jnp.float32) for _ in range(NUM_TABLES)
    )

    @functools.partial(
        pl.kernel,
        out_type=out_type,
        mesh=mesh,
        scratch_types=[
            pltpu.VMEM((CHUNK,), jnp.int32),
            pltpu.VMEM((CHUNK, EMB), jnp.float32),
            pltpu.SemaphoreType.DMA,
        ],
    )
    def body(*refs):
        idx_refs = refs[0:NUM_TABLES]
        tab_refs = refs[NUM_TABLES:2 * NUM_TABLES]
        out_refs = refs[2 * NUM_TABLES:3 * NUM_TABLES]
        idx_v, rows_v, sem = refs[3 * NUM_TABLES:]

        wid = lax.axis_index("s") * NC + lax.axis_index("c")
        base = wid * PER_W

        for t in range(NUM_TABLES):
            @pl.loop(0, NCHUNK)
            def _(g, _t=t):
                off = base + g * CHUNK
                pltpu.sync_copy(idx_refs[_t].at[pl.ds(off, CHUNK)], idx_v)
                pltpu.async_copy(tab_refs[_t].at[idx_v], rows_v, sem).wait()
                pltpu.sync_copy(rows_v, out_refs[_t].at[pl.ds(off, CHUNK)])

    return body(*args)


def kernel(A1, A2, A3, B1, B2, B3, peptide,
           W_a1, W_a2, W_a3, W_b1, W_b2, W_b3, W_peptide):
    idx = [x.reshape(N).astype(jnp.int32)
           for x in (A1, A2, A3, B1, B2, B3, peptide)]
    tabs = [W_a1, W_a2, W_a3, W_b1, W_b2, W_b3, W_peptide]
    outs = _sc_lookup(*idx, *tabs)
    return tuple(o.reshape(B, L, EMB) for o in outs)


# speedup vs baseline: 1.5550x; 1.0676x over previous
import functools

import jax
import jax.numpy as jnp
from jax import lax
from jax.experimental import pallas as pl
from jax.experimental.pallas import tpu as pltpu
from jax.experimental.pallas import tpu_sc as plsc

VOCAB = 33
EMB = 128
B = 16384
L = 20
N = B * L
NUM_TABLES = 7

NC = 2
NS = 16
NW = NC * NS
PER_W = N // NW
CHUNK = 128
NCHUNK = PER_W // CHUNK
NBUF = 5
LAG = 2
NROUND = NCHUNK // NBUF


def _sc_lookup(*args):
    mesh = plsc.VectorSubcoreMesh(core_axis_name="c", subcore_axis_name="s")
    out_type = tuple(
        jax.ShapeDtypeStruct((N, EMB), jnp.float32) for _ in range(NUM_TABLES)
    )

    scratch = [pltpu.VMEM((NCHUNK, CHUNK), jnp.int32)]
    scratch += [pltpu.VMEM((CHUNK, EMB), jnp.float32) for _ in range(NBUF)]
    scratch += [pltpu.SemaphoreType.DMA for _ in range(2 * NBUF)]

    @functools.partial(
        pl.kernel,
        out_type=out_type,
        mesh=mesh,
        scratch_types=scratch,
    )
    def body(*refs):
        idx_refs = refs[0:NUM_TABLES]
        tab_refs = refs[NUM_TABLES:2 * NUM_TABLES]
        out_refs = refs[2 * NUM_TABLES:3 * NUM_TABLES]
        rest = refs[3 * NUM_TABLES:]
        idx_v = rest[0]
        rows = rest[1:1 + NBUF]
        gsem = rest[1 + NBUF:1 + 2 * NBUF]
        osem = rest[1 + 2 * NBUF:1 + 3 * NBUF]

        wid = lax.axis_index("s") * NC + lax.axis_index("c")
        base = wid * PER_W
        ibase = wid * NCHUNK

        def start_gather(t, c, b):
            return pltpu.async_copy(tab_refs[t].at[idx_v.at[c]], rows[b],
                                    gsem[b])

        def wait_gather(t, c, b):
            pltpu.make_async_copy(tab_refs[t].at[idx_v.at[c]], rows[b],
                                  gsem[b]).wait()

        def start_out(t, c, b):
            return pltpu.async_copy(
                rows[b], out_refs[t].at[pl.ds(base + c * CHUNK, CHUNK)],
                osem[b])

        def wait_out(t, c, b):
            pltpu.make_async_copy(
                rows[b], out_refs[t].at[pl.ds(base + c * CHUNK, CHUNK)],
                osem[b]).wait()

        for t in range(NUM_TABLES):
            pltpu.sync_copy(idx_refs[t].at[pl.ds(ibase, NCHUNK)], idx_v)

            for b in range(NBUF):
                if t > 0:
                    wait_out(t - 1, NCHUNK - NBUF + b, b)
                start_gather(t, b, b)
                if b >= LAG:
                    bo = b - LAG
                    wait_gather(t, bo, bo)
                    start_out(t, bo, bo)

            @pl.loop(1, NROUND)
            def _(r, _t=t):
                for b in range(NBUF):
                    s = r * NBUF + b
                    wait_out(_t, s - NBUF, b)
                    start_gather(_t, s, b)
                    bo = (b - LAG) % NBUF
                    wait_gather(_t, s - LAG, bo)
                    start_out(_t, s - LAG, bo)

            for i in range(LAG):
                c = NCHUNK - LAG + i
                b = c % NBUF
                wait_gather(t, c, b)
                start_out(t, c, b)

        for b in range(NBUF):
            wait_out(NUM_TABLES - 1, NCHUNK - NBUF + b, b)

    return body(*args)


def kernel(A1, A2, A3, B1, B2, B3, peptide,
           W_a1, W_a2, W_a3, W_b1, W_b2, W_b3, W_peptide):
    idx = [x.reshape(N // CHUNK, CHUNK).astype(jnp.int32)
           for x in (A1, A2, A3, B1, B2, B3, peptide)]
    tabs = [W_a1, W_a2, W_a3, W_b1, W_b2, W_b3, W_peptide]
    outs = _sc_lookup(*idx, *tabs)
    return tuple(o.reshape(B, L, EMB) for o in outs)


# speedup vs baseline: 1.7895x; 1.1508x over previous
import functools

import jax
import jax.numpy as jnp
from jax import lax
from jax.experimental import pallas as pl
from jax.experimental.pallas import tpu as pltpu
from jax.experimental.pallas import tpu_sc as plsc

VOCAB = 33
EMB = 128
B = 16384
L = 20
N = B * L
NUM_TABLES = 7

NC = 2
NS = 16
NW = NC * NS
BPW = B // NW
CB = 4
CHUNK = CB * L
NCHUNK = BPW // CB
NBUF = 4
LAG = 2
NROUND = NCHUNK // NBUF


def _sc_lookup(*args):
    mesh = plsc.VectorSubcoreMesh(core_axis_name="c", subcore_axis_name="s")
    out_type = tuple(
        jax.ShapeDtypeStruct((B, L, EMB), jnp.float32)
        for _ in range(NUM_TABLES)
    )

    scratch = [pltpu.VMEM((NCHUNK, CHUNK), jnp.int32)]
    scratch += [pltpu.VMEM((CHUNK, EMB), jnp.float32) for _ in range(NBUF)]
    scratch += [pltpu.SemaphoreType.DMA for _ in range(2 * NBUF)]

    @functools.partial(
        pl.kernel,
        out_type=out_type,
        mesh=mesh,
        scratch_types=scratch,
    )
    def body(*refs):
        idx_refs = refs[0:NUM_TABLES]
        tab_refs = refs[NUM_TABLES:2 * NUM_TABLES]
        out_refs = refs[2 * NUM_TABLES:3 * NUM_TABLES]
        rest = refs[3 * NUM_TABLES:]
        idx_v = rest[0]
        rows = rest[1:1 + NBUF]
        gsem = rest[1 + NBUF:1 + 2 * NBUF]
        osem = rest[1 + 2 * NBUF:1 + 3 * NBUF]

        wid = lax.axis_index("s") * NC + lax.axis_index("c")
        bbase = wid * BPW
        ibase = wid * NCHUNK

        def start_gather(t, c, b):
            return pltpu.async_copy(tab_refs[t].at[idx_v.at[c]], rows[b],
                                    gsem[b])

        def wait_gather(t, c, b):
            pltpu.make_async_copy(tab_refs[t].at[idx_v.at[c]], rows[b],
                                  gsem[b]).wait()

        def start_out(t, c, b):
            for j in range(CB):
                pltpu.async_copy(rows[b].at[pl.ds(j * L, L)],
                                 out_refs[t].at[bbase + c * CB + j],
                                 osem[b])

        def wait_out(t, c, b):
            for j in range(CB):
                pltpu.make_async_copy(rows[b].at[pl.ds(j * L, L)],
                                      out_refs[t].at[bbase + c * CB + j],
                                      osem[b]).wait()

        for t in range(NUM_TABLES):
            pltpu.sync_copy(idx_refs[t].at[pl.ds(ibase, NCHUNK)], idx_v)

            for b in range(NBUF):
                if t > 0:
                    wait_out(t - 1, NCHUNK - NBUF + b, b)
                start_gather(t, b, b)
                if b >= LAG:
                    bo = b - LAG
                    wait_gather(t, bo, bo)
                    start_out(t, bo, bo)

            @pl.loop(1, NROUND)
            def _(r, _t=t):
                for b in range(NBUF):
                    s = r * NBUF + b
                    wait_out(_t, s - NBUF, b)
                    start_gather(_t, s, b)
                    bo = (b - LAG) % NBUF
                    wait_gather(_t, s - LAG, bo)
                    start_out(_t, s - LAG, bo)

            for i in range(LAG):
                c = NCHUNK - LAG + i
                b = c % NBUF
                wait_gather(t, c, b)
                start_out(t, c, b)

        for b in range(NBUF):
            wait_out(NUM_TABLES - 1, NCHUNK - NBUF + b, b)

    return body(*args)


def kernel(A1, A2, A3, B1, B2, B3, peptide,
           W_a1, W_a2, W_a3, W_b1, W_b2, W_b3, W_peptide):
    idx = [x.reshape(N // CHUNK, CHUNK).astype(jnp.int32)
           for x in (A1, A2, A3, B1, B2, B3, peptide)]
    tabs = [W_a1, W_a2, W_a3, W_b1, W_b2, W_b3, W_peptide]
    return _sc_lookup(*idx, *tabs)


# speedup vs baseline: 1.8244x; 1.0195x over previous
import functools

import jax
import jax.numpy as jnp
from jax import lax
from jax.experimental import pallas as pl
from jax.experimental.pallas import tpu as pltpu
from jax.experimental.pallas import tpu_sc as plsc

VOCAB = 33
EMB = 128
B = 16384
L = 20
N = B * L
NUM_TABLES = 7

NC = 2
NS = 16
NW = NC * NS
BPW = B // NW
CB = 4
CHUNK = CB * L
NCHUNK = BPW // CB
NBUF = 8
LAG = 4
NROUND = NCHUNK // NBUF


def _sc_lookup(*args):
    mesh = plsc.VectorSubcoreMesh(core_axis_name="c", subcore_axis_name="s")
    out_type = tuple(
        jax.ShapeDtypeStruct((B, L, EMB), jnp.float32)
        for _ in range(NUM_TABLES)
    )

    scratch = [pltpu.VMEM((NCHUNK, CHUNK), jnp.int32)]
    scratch += [pltpu.VMEM((CHUNK, EMB), jnp.float32) for _ in range(NBUF)]
    scratch += [pltpu.SemaphoreType.DMA for _ in range(2 * NBUF)]

    @functools.partial(
        pl.kernel,
        out_type=out_type,
        mesh=mesh,
        scratch_types=scratch,
    )
    def body(*refs):
        idx_refs = refs[0:NUM_TABLES]
        tab_refs = refs[NUM_TABLES:2 * NUM_TABLES]
        out_refs = refs[2 * NUM_TABLES:3 * NUM_TABLES]
        rest = refs[3 * NUM_TABLES:]
        idx_v = rest[0]
        rows = rest[1:1 + NBUF]
        gsem = rest[1 + NBUF:1 + 2 * NBUF]
        osem = rest[1 + 2 * NBUF:1 + 3 * NBUF]

        wid = lax.axis_index("s") * NC + lax.axis_index("c")
        bbase = wid * BPW
        ibase = wid * NCHUNK

        def start_gather(t, c, b):
            return pltpu.async_copy(tab_refs[t].at[idx_v.at[c]], rows[b],
                                    gsem[b])

        def wait_gather(t, c, b):
            pltpu.make_async_copy(tab_refs[t].at[idx_v.at[c]], rows[b],
                                  gsem[b]).wait()

        def start_out(t, c, b):
            for j in range(CB):
                pltpu.async_copy(rows[b].at[pl.ds(j * L, L)],
                                 out_refs[t].at[bbase + c * CB + j],
                                 osem[b])

        def wait_out(t, c, b):
            for j in range(CB):
                pltpu.make_async_copy(rows[b].at[pl.ds(j * L, L)],
                                      out_refs[t].at[bbase + c * CB + j],
                                      osem[b]).wait()

        for t in range(NUM_TABLES):
            pltpu.sync_copy(idx_refs[t].at[pl.ds(ibase, NCHUNK)], idx_v)

            for b in range(NBUF):
                if t > 0:
                    wait_out(t - 1, NCHUNK - NBUF + b, b)
                start_gather(t, b, b)
                if b >= LAG:
                    bo = b - LAG
                    wait_gather(t, bo, bo)
                    start_out(t, bo, bo)

            @pl.loop(1, NROUND)
            def _(r, _t=t):
                for b in range(NBUF):
                    s = r * NBUF + b
                    wait_out(_t, s - NBUF, b)
                    start_gather(_t, s, b)
                    bo = (b - LAG) % NBUF
                    wait_gather(_t, s - LAG, bo)
                    start_out(_t, s - LAG, bo)

            for i in range(LAG):
                c = NCHUNK - LAG + i
                b = c % NBUF
                wait_gather(t, c, b)
                start_out(t, c, b)

        for b in range(NBUF):
            wait_out(NUM_TABLES - 1, NCHUNK - NBUF + b, b)

    return body(*args)


def kernel(A1, A2, A3, B1, B2, B3, peptide,
           W_a1, W_a2, W_a3, W_b1, W_b2, W_b3, W_peptide):
    idx = [x.reshape(N // CHUNK, CHUNK).astype(jnp.int32)
           for x in (A1, A2, A3, B1, B2, B3, peptide)]
    tabs = [W_a1, W_a2, W_a3, W_b1, W_b2, W_b3, W_peptide]
    return _sc_lookup(*idx, *tabs)


# speedup vs baseline: 4.5394x; 2.4881x over previous
import functools

import jax
import jax.numpy as jnp
from jax import lax
from jax.experimental import pallas as pl
from jax.experimental.pallas import tpu as pltpu
from jax.experimental.pallas import tpu_sc as plsc

VOCAB = 33
EMB = 128
B = 16384
L = 20
N = B * L
NUM_TABLES = 7
VOCAB_PAD = 40

NC = 2
NS = 16
NW = NC * NS
BPW = B // NW
CB = 4
CHUNK = CB * L
NCHUNK = BPW // CB
NBUF = 8
LAG = 4
NROUND = NCHUNK // NBUF


def _sc_lookup(*args):
    mesh = plsc.VectorSubcoreMesh(core_axis_name="c", subcore_axis_name="s")
    out_type = tuple(
        jax.ShapeDtypeStruct((B, L, EMB), jnp.float32)
        for _ in range(NUM_TABLES)
    )

    scratch = [pltpu.VMEM((NCHUNK, CHUNK), jnp.int32)]
    scratch += [pltpu.VMEM((CHUNK, EMB), jnp.float32) for _ in range(NBUF)]
    scratch += [pltpu.SemaphoreType.DMA for _ in range(2 * NBUF)]

    @functools.partial(
        pl.kernel,
        out_type=out_type,
        mesh=mesh,
        scratch_types=scratch,
    )
    def body(*refs):
        idx_refs = refs[0:NUM_TABLES]
        tab_refs = refs[NUM_TABLES:2 * NUM_TABLES]
        out_refs = refs[2 * NUM_TABLES:3 * NUM_TABLES]
        rest = refs[3 * NUM_TABLES:]
        idx_v = rest[0]
        rows = rest[1:1 + NBUF]
        gsem = rest[1 + NBUF:1 + 2 * NBUF]
        osem = rest[1 + 2 * NBUF:1 + 3 * NBUF]

        wid = lax.axis_index("s") * NC + lax.axis_index("c")
        bbase = wid * BPW
        ibase = wid * NCHUNK

        def start_gather(t, c, b):
            src = tab_refs[t].at[pl.ds(wid * VOCAB_PAD, VOCAB_PAD)].at[idx_v.at[c]]
            return pltpu.async_copy(src, rows[b], gsem[b])

        def wait_gather(t, c, b):
            src = tab_refs[t].at[pl.ds(wid * VOCAB_PAD, VOCAB_PAD)].at[idx_v.at[c]]
            pltpu.make_async_copy(src, rows[b], gsem[b]).wait()

        def start_out(t, c, b):
            for j in range(CB):
                pltpu.async_copy(rows[b].at[pl.ds(j * L, L)],
                                 out_refs[t].at[bbase + c * CB + j],
                                 osem[b])

        def wait_out(t, c, b):
            for j in range(CB):
                pltpu.make_async_copy(rows[b].at[pl.ds(j * L, L)],
                                      out_refs[t].at[bbase + c * CB + j],
                                      osem[b]).wait()

        for t in range(NUM_TABLES):
            pltpu.sync_copy(idx_refs[t].at[pl.ds(ibase, NCHUNK)], idx_v)

            for b in range(NBUF):
                if t > 0:
                    wait_out(t - 1, NCHUNK - NBUF + b, b)
                start_gather(t, b, b)
                if b >= LAG:
                    bo = b - LAG
                    wait_gather(t, bo, bo)
                    start_out(t, bo, bo)

            @pl.loop(1, NROUND)
            def _(r, _t=t):
                for b in range(NBUF):
                    s = r * NBUF + b
                    wait_out(_t, s - NBUF, b)
                    start_gather(_t, s, b)
                    bo = (b - LAG) % NBUF
                    wait_gather(_t, s - LAG, bo)
                    start_out(_t, s - LAG, bo)

            for i in range(LAG):
                c = NCHUNK - LAG + i
                b = c % NBUF
                wait_gather(t, c, b)
                start_out(t, c, b)

        for b in range(NBUF):
            wait_out(NUM_TABLES - 1, NCHUNK - NBUF + b, b)

    return body(*args)


def kernel(A1, A2, A3, B1, B2, B3, peptide,
           W_a1, W_a2, W_a3, W_b1, W_b2, W_b3, W_peptide):
    idx = [x.reshape(N // CHUNK, CHUNK).astype(jnp.int32)
           for x in (A1, A2, A3, B1, B2, B3, peptide)]
    tabs = [jnp.tile(jnp.pad(w, ((0, VOCAB_PAD - VOCAB), (0, 0))), (NW, 1))
            for w in (W_a1, W_a2, W_a3, W_b1, W_b2, W_b3, W_peptide)]
    return _sc_lookup(*idx, *tabs)


# speedup vs baseline: 4.6337x; 1.0208x over previous
import functools

import jax
import jax.numpy as jnp
from jax import lax
from jax.experimental import pallas as pl
from jax.experimental.pallas import tpu as pltpu
from jax.experimental.pallas import tpu_sc as plsc

VOCAB = 33
EMB = 128
B = 16384
L = 20
N = B * L
NUM_TABLES = 7
VOCAB_PAD = 40

NC = 2
NS = 16
NW = NC * NS
BPW = B // NW
CB = 4
CHUNK = CB * L
NCHUNK = BPW // CB
NBUF = 8
LAG = 4
NROUND = NCHUNK // NBUF


def _sc_lookup(idx_all, tab_all):
    mesh = plsc.VectorSubcoreMesh(core_axis_name="c", subcore_axis_name="s")
    out_type = tuple(
        jax.ShapeDtypeStruct((B, L, EMB), jnp.float32)
        for _ in range(NUM_TABLES)
    )

    scratch = [pltpu.VMEM((NCHUNK, CHUNK), jnp.int32) for _ in range(2)]
    scratch += [pltpu.VMEM((CHUNK, EMB), jnp.float32) for _ in range(NBUF)]
    scratch += [pltpu.SemaphoreType.DMA for _ in range(2 + 2 * NBUF)]

    @functools.partial(
        pl.kernel,
        out_type=out_type,
        mesh=mesh,
        scratch_types=scratch,
    )
    def body(idx_ref, tab_ref, *refs):
        out_refs = refs[0:NUM_TABLES]
        rest = refs[NUM_TABLES:]
        idx_v = rest[0:2]
        rows = rest[2:2 + NBUF]
        isem = rest[2 + NBUF:4 + NBUF]
        gsem = rest[4 + NBUF:4 + 2 * NBUF]
        osem = rest[4 + 2 * NBUF:4 + 3 * NBUF]

        wid = lax.axis_index("s") * NC + lax.axis_index("c")
        bbase = wid * BPW
        ibase = wid * NCHUNK

        def start_idx(t):
            return pltpu.async_copy(
                idx_ref.at[t].at[pl.ds(ibase, NCHUNK)], idx_v[t % 2],
                isem[t % 2])

        def wait_idx(t):
            pltpu.make_async_copy(
                idx_ref.at[t].at[pl.ds(ibase, NCHUNK)], idx_v[t % 2],
                isem[t % 2]).wait()

        def gather_src(t, c):
            return tab_ref.at[t].at[pl.ds(wid * VOCAB_PAD, VOCAB_PAD)].at[
                idx_v[t % 2].at[c]]

        def start_gather(t, c, b):
            return pltpu.async_copy(gather_src(t, c), rows[b], gsem[b])

        def wait_gather(t, c, b):
            pltpu.make_async_copy(gather_src(t, c), rows[b], gsem[b]).wait()

        def start_out(t, c, b):
            pltpu.async_copy(rows[b].reshape(CB, L, EMB),
                             out_refs[t].at[pl.ds(bbase + c * CB, CB)],
                             osem[b])

        def wait_out(t, c, b):
            pltpu.make_async_copy(rows[b].reshape(CB, L, EMB),
                                  out_refs[t].at[pl.ds(bbase + c * CB, CB)],
                                  osem[b]).wait()

        start_idx(0)
        for t in range(NUM_TABLES):
            wait_idx(t)
            if t + 1 < NUM_TABLES:
                start_idx(t + 1)

            for b in range(NBUF):
                if t > 0:
                    wait_out(t - 1, NCHUNK - NBUF + b, b)
                start_gather(t, b, b)
                if b >= LAG:
                    bo = b - LAG
                    wait_gather(t, bo, bo)
                    start_out(t, bo, bo)

            @pl.loop(1, NROUND)
            def _(r, _t=t):
                for b in range(NBUF):
                    s = r * NBUF + b
                    wait_out(_t, s - NBUF, b)
                    start_gather(_t, s, b)
                    bo = (b - LAG) % NBUF
                    wait_gather(_t, s - LAG, bo)
                    start_out(_t, s - LAG, bo)

            for i in range(LAG):
                c = NCHUNK - LAG + i
                b = c % NBUF
                wait_gather(t, c, b)
                start_out(t, c, b)

        for b in range(NBUF):
            wait_out(NUM_TABLES - 1, NCHUNK - NBUF + b, b)

    return body(idx_all, tab_all)


def kernel(A1, A2, A3, B1, B2, B3, peptide,
           W_a1, W_a2, W_a3, W_b1, W_b2, W_b3, W_peptide):
    idx_all = jnp.stack(
        [x.reshape(N // CHUNK, CHUNK).astype(jnp.int32)
         for x in (A1, A2, A3, B1, B2, B3, peptide)])
    tab_all = jnp.stack(
        [jnp.tile(jnp.pad(w, ((0, VOCAB_PAD - VOCAB), (0, 0))), (NW, 1))
         for w in (W_a1, W_a2, W_a3, W_b1, W_b2, W_b3, W_peptide)])
    return _sc_lookup(idx_all, tab_all)


# speedup vs baseline: 4.7127x; 1.0170x over previous
import functools

import jax
import jax.numpy as jnp
from jax import lax
from jax.experimental import pallas as pl
from jax.experimental.pallas import tpu as pltpu
from jax.experimental.pallas import tpu_sc as plsc

VOCAB = 33
EMB = 128
B = 16384
L = 20
N = B * L
NUM_TABLES = 7
VOCAB_PAD = 40

NC = 2
NS = 16
NW = NC * NS
BPW = B // NW
CB = 4
CHUNK = CB * L
NCHUNK = BPW // CB
NBUF = 8
LAG = 2
NROUND = NCHUNK // NBUF


def _sc_lookup(idx_all, tab_all):
    mesh = plsc.VectorSubcoreMesh(core_axis_name="c", subcore_axis_name="s")
    out_type = tuple(
        jax.ShapeDtypeStruct((B, L, EMB), jnp.float32)
        for _ in range(NUM_TABLES)
    )

    scratch = [pltpu.VMEM((NCHUNK, CHUNK), jnp.int32) for _ in range(2)]
    scratch += [pltpu.VMEM((CHUNK, EMB), jnp.float32) for _ in range(NBUF)]
    scratch += [pltpu.SemaphoreType.DMA for _ in range(2 + 2 * NBUF)]

    @functools.partial(
        pl.kernel,
        out_type=out_type,
        mesh=mesh,
        scratch_types=scratch,
    )
    def body(idx_ref, tab_ref, *refs):
        out_refs = refs[0:NUM_TABLES]
        rest = refs[NUM_TABLES:]
        idx_v = rest[0:2]
        rows = rest[2:2 + NBUF]
        isem = rest[2 + NBUF:4 + NBUF]
        gsem = rest[4 + NBUF:4 + 2 * NBUF]
        osem = rest[4 + 2 * NBUF:4 + 3 * NBUF]

        wid = lax.axis_index("s") * NC + lax.axis_index("c")
        bbase = wid * BPW
        ibase = wid * NCHUNK

        def start_idx(t):
            return pltpu.async_copy(
                idx_ref.at[t].at[pl.ds(ibase, NCHUNK)], idx_v[t % 2],
                isem[t % 2])

        def wait_idx(t):
            pltpu.make_async_copy(
                idx_ref.at[t].at[pl.ds(ibase, NCHUNK)], idx_v[t % 2],
                isem[t % 2]).wait()

        def gather_src(t, c):
            return tab_ref.at[t].at[pl.ds(wid * VOCAB_PAD, VOCAB_PAD)].at[
                idx_v[t % 2].at[c]]

        def start_gather(t, c, b):
            return pltpu.async_copy(gather_src(t, c), rows[b], gsem[b])

        def wait_gather(t, c, b):
            pltpu.make_async_copy(gather_src(t, c), rows[b], gsem[b]).wait()

        def start_out(t, c, b):
            pltpu.async_copy(rows[b].reshape(CB, L, EMB),
                             out_refs[t].at[pl.ds(bbase + c * CB, CB)],
                             osem[b])

        def wait_out(t, c, b):
            pltpu.make_async_copy(rows[b].reshape(CB, L, EMB),
                                  out_refs[t].at[pl.ds(bbase + c * CB, CB)],
                                  osem[b]).wait()

        start_idx(0)
        for t in range(NUM_TABLES):
            wait_idx(t)
            if t + 1 < NUM_TABLES:
                start_idx(t + 1)

            for b in range(NBUF):
                if t > 0:
                    wait_out(t - 1, NCHUNK - NBUF + b, b)
                start_gather(t, b, b)
                if b >= LAG:
                    bo = b - LAG
                    wait_gather(t, bo, bo)
                    start_out(t, bo, bo)

            @pl.loop(1, NROUND)
            def _(r, _t=t):
                for b in range(NBUF):
                    s = r * NBUF + b
                    wait_out(_t, s - NBUF, b)
                    start_gather(_t, s, b)
                    bo = (b - LAG) % NBUF
                    wait_gather(_t, s - LAG, bo)
                    start_out(_t, s - LAG, bo)

            for i in range(LAG):
                c = NCHUNK - LAG + i
                b = c % NBUF
                wait_gather(t, c, b)
                start_out(t, c, b)

        for b in range(NBUF):
            wait_out(NUM_TABLES - 1, NCHUNK - NBUF + b, b)

    return body(idx_all, tab_all)


def kernel(A1, A2, A3, B1, B2, B3, peptide,
           W_a1, W_a2, W_a3, W_b1, W_b2, W_b3, W_peptide):
    idx_all = jnp.stack(
        [x.reshape(N // CHUNK, CHUNK).astype(jnp.int32)
         for x in (A1, A2, A3, B1, B2, B3, peptide)])
    tab_all = jnp.stack(
        [jnp.tile(jnp.pad(w, ((0, VOCAB_PAD - VOCAB), (0, 0))), (NW, 1))
         for w in (W_a1, W_a2, W_a3, W_b1, W_b2, W_b3, W_peptide)])
    return _sc_lookup(idx_all, tab_all)
